# baseline (speedup 1.0000x reference)
import functools

import jax
import jax.numpy as jnp
from jax import lax
from jax.experimental import pallas as pl
from jax.experimental.pallas import tpu as pltpu
from jax.experimental.pallas import tpu_sc as plsc

N = 10000
E = 320000
D = 128
NUM_GATE = 29
ACT_H = 256
CRIT_H = 128
ADIM = 512

_BLK = 1000

_NC = 2
_NS = 16
_NW = _NC * _NS
_CH = 256
_CPW = 40
_EPAD = _NW * _CH * _CPW
_NROWS = 10112
_RPT = _NROWS // _NS
_NTPAD = 10112


def _worker_id():
    c = lax.axis_index("c")
    s = lax.axis_index("s")
    return c, s, c * _NS + s


def _zero_vmem_2d(ref, nrows, width):
    zeros16 = jnp.zeros((16,), jnp.float32)

    def row(i, _):
        for j in range(width // 16):
            ref[i, pl.ds(j * 16, 16)] = zeros16
        return 0

    lax.fori_loop(0, nrows, row, 0)


def _zero_spmem_slice_async(zb_ref, sh_ref, base, total, sem):
    cps = []
    off = 0
    while off < total:
        n = min(128, total - off)
        cps.append(pltpu.async_copy(zb_ref.at[pl.ds(0, n)],
                                    sh_ref.at[pl.ds(base + off, n)], sem))
        off += n
    return cps


def _sc_counts_body(nt_hbm, ei_hbm, out_hbm, nt_v,
                    r0_v, r1_v, d0_v, d1_v, oh_v, idx_v, counts_sh,
                    semi, semr0, semr1, sems, semz):
    c, s, w = _worker_id()
    wbase = w * (_CPW * _CH)
    lane = lax.broadcasted_iota(jnp.int32, (16,), 0)
    ones16 = jnp.full((16,), 1.0, jnp.float32)
    zeros16 = jnp.zeros((16,), jnp.float32)
    rv = (r0_v, r1_v)
    dv = (d0_v, d1_v)
    semr = (semr0, semr1)

    def start_idx(t, ci):
        base = wbase + ci * _CH
        pltpu.async_copy(ei_hbm.at[0, pl.ds(base, _CH)], rv[t], semr[t])
        pltpu.async_copy(ei_hbm.at[1, pl.ds(base, _CH)], dv[t], semr[t])

    def wait_idx(t):
        pltpu.make_async_copy(ei_hbm.at[0, pl.ds(0, _CH)], rv[t], semr[t]).wait()
        pltpu.make_async_copy(ei_hbm.at[1, pl.ds(0, _CH)], dv[t], semr[t]).wait()

    cp_nt = pltpu.async_copy(nt_hbm, nt_v.at[pl.ds(0, N)], semi)
    start_idx(0, 0)
    start_idx(1, 1)
    _zero_vmem_2d(oh_v, 128, D)
    zcps = _zero_spmem_slice_async(oh_v, counts_sh, s * _RPT, _RPT, semz)
    _zero_vmem_2d(oh_v.at[pl.ds(128, _CH - 128)], _CH - 128, D)
    cp_nt.wait()

    def build(t):
        for k in range(_CH // 16):
            s16 = rv[t][pl.ds(k * 16, 16)]
            t16 = plsc.load_gather(nt_v, [s16])
            row16 = lane + (k * 16)
            plsc.store_scatter(oh_v, [row16, t16], ones16)
            idx_v[pl.ds(k * 16, 16)] = t16

    def rezero():
        for k in range(_CH // 16):
            t16 = idx_v[pl.ds(k * 16, 16)]
            row16 = lane + (k * 16)
            plsc.store_scatter(oh_v, [row16, t16], zeros16)

    for cp in zcps:
        cp.wait()
    plsc.subcore_barrier()

    def pair(m, _):
        for t in (0, 1):
            ci = 2 * m + t
            wait_idx(t)
            build(t)
            pltpu.async_copy(oh_v, counts_sh.at[dv[t]], sems, add=True)
            pltpu.make_async_copy(oh_v, counts_sh.at[dv[t]], sems).wait()
            rezero()

            @pl.when(ci + 2 < _CPW)
            def _():
                start_idx(t, ci + 2)
        return 0

    lax.fori_loop(0, _CPW // 2, pair, 0)

    plsc.subcore_barrier()
    row0 = s * _RPT
    pltpu.sync_copy(counts_sh.at[pl.ds(row0, _RPT)],
                    out_hbm.at[c, pl.ds(row0, _RPT)])


def _sc_counts(node_types, ei_pad):
    mesh = plsc.VectorSubcoreMesh(core_axis_name="c", subcore_axis_name="s")
    f = pl.kernel(
        _sc_counts_body,
        out_type=jax.ShapeDtypeStruct((_NC, _NROWS, D), jnp.float32),
        mesh=mesh,
        compiler_params=pltpu.CompilerParams(needs_layout_passes=False),
        scratch_types=[
            pltpu.VMEM((_NTPAD,), jnp.int32),
            pltpu.VMEM((_CH,), jnp.int32),
            pltpu.VMEM((_CH,), jnp.int32),
            pltpu.VMEM((_CH,), jnp.int32),
            pltpu.VMEM((_CH,), jnp.int32),
            pltpu.VMEM((_CH, D), jnp.float32),
            pltpu.VMEM((_CH,), jnp.int32),
            pltpu.VMEM_SHARED((_NROWS, D), jnp.float32),
            pltpu.SemaphoreType.DMA,
            pltpu.SemaphoreType.DMA,
            pltpu.SemaphoreType.DMA,
            pltpu.SemaphoreType.DMA,
            pltpu.SemaphoreType.DMA,
        ],
    )
    return f(node_types, ei_pad)


def _sc_agg_body(x_hbm, ei_hbm, out_hbm,
                 r0_v, r1_v, d0_v, d1_v, rows_v, agg_sh,
                 semr0, semr1, semg, sems, semz):
    c, s, w = _worker_id()
    wbase = w * (_CPW * _CH)
    rv = (r0_v, r1_v)
    dv = (d0_v, d1_v)
    semr = (semr0, semr1)

    def start_idx(t, ci):
        base = wbase + ci * _CH
        pltpu.async_copy(ei_hbm.at[0, pl.ds(base, _CH)], rv[t], semr[t])
        pltpu.async_copy(ei_hbm.at[1, pl.ds(base, _CH)], dv[t], semr[t])

    def wait_idx(t):
        pltpu.make_async_copy(ei_hbm.at[0, pl.ds(0, _CH)], rv[t], semr[t]).wait()
        pltpu.make_async_copy(ei_hbm.at[1, pl.ds(0, _CH)], dv[t], semr[t]).wait()

    start_idx(0, 0)
    start_idx(1, 1)
    _zero_vmem_2d(rows_v, 128, D)
    zcps = _zero_spmem_slice_async(rows_v, agg_sh, s * _RPT, _RPT, semz)
    for cp in zcps:
        cp.wait()
    plsc.subcore_barrier()

    def pair(m, _):
        for t in (0, 1):
            ci = 2 * m + t
            wait_idx(t)
            pltpu.async_copy(x_hbm.at[rv[t]], rows_v, semg)
            pltpu.make_async_copy(x_hbm.at[rv[t]], rows_v, semg).wait()
            pltpu.async_copy(rows_v, agg_sh.at[dv[t]], sems, add=True)
            pltpu.make_async_copy(rows_v, agg_sh.at[dv[t]], sems).wait()

            @pl.when(ci + 2 < _CPW)
            def _():
                start_idx(t, ci + 2)
        return 0

    lax.fori_loop(0, _CPW // 2, pair, 0)

    plsc.subcore_barrier()
    row0 = s * _RPT
    pltpu.sync_copy(agg_sh.at[pl.ds(row0, _RPT)],
                    out_hbm.at[c, pl.ds(row0, _RPT)])


def _sc_agg(x, ei_pad):
    mesh = plsc.VectorSubcoreMesh(core_axis_name="c", subcore_axis_name="s")
    f = pl.kernel(
        _sc_agg_body,
        out_type=jax.ShapeDtypeStruct((_NC, _NROWS, D), jnp.float32),
        mesh=mesh,
        compiler_params=pltpu.CompilerParams(needs_layout_passes=False),
        scratch_types=[
            pltpu.VMEM((_CH,), jnp.int32),
            pltpu.VMEM((_CH,), jnp.int32),
            pltpu.VMEM((_CH,), jnp.int32),
            pltpu.VMEM((_CH,), jnp.int32),
            pltpu.VMEM((_CH, D), jnp.float32),
            pltpu.VMEM_SHARED((_NROWS, D), jnp.float32),
            pltpu.SemaphoreType.DMA,
            pltpu.SemaphoreType.DMA,
            pltpu.SemaphoreType.DMA,
            pltpu.SemaphoreType.DMA,
            pltpu.SemaphoreType.DMA,
        ],
    )
    return f(x, ei_pad)


def _layer0_body(nt_ref, c0_ref, c1_ref, tab_ref, wt_ref, wb_ref, b_ref,
                 o_ref, inv_ref):
    counts = c0_ref[...] + c1_ref[...]
    deg = jnp.maximum(jnp.sum(counts, axis=1), 1.0)
    inv = 1.0 / deg
    tab = tab_ref[...]
    types_row = nt_ref[0, ...]
    gates = lax.broadcasted_iota(jnp.int32, (D, _BLK), 0)
    onehot_t = (gates == types_row).astype(jnp.float32)
    x0 = lax.dot_general(onehot_t, tab, (((0,), (0,)), ((), ())),
                         preferred_element_type=jnp.float32)
    agg0 = jnp.dot(counts, tab, preferred_element_type=jnp.float32)
    agg0 = agg0 * inv[:, None]
    acc = jnp.dot(x0, wt_ref[...], preferred_element_type=jnp.float32)
    acc += jnp.dot(agg0, wb_ref[...], preferred_element_type=jnp.float32)
    o_ref[...] = jnp.maximum(acc + b_ref[...], 0.0)
    inv_ref[...] = inv[:, None]


def _layer0(node_types, counts, embed_table, W, b):
    wt, wb = W[:D], W[D:]
    tab = jnp.zeros((D, D), jnp.float32).at[:NUM_GATE].set(embed_table)
    nt3 = node_types.astype(jnp.int32).reshape(N // _BLK, 1, _BLK)
    return pl.pallas_call(
        _layer0_body,
        grid=(N // _BLK,),
        in_specs=[
            pl.BlockSpec((1, 1, _BLK), lambda i: (i, 0, 0)),
            pl.BlockSpec((_BLK, D), lambda i: (i, 0)),
            pl.BlockSpec((_BLK, D), lambda i: (i, 0)),
            pl.BlockSpec((D, D), lambda i: (0, 0)),
            pl.BlockSpec((D, D), lambda i: (0, 0)),
            pl.BlockSpec((D, D), lambda i: (0, 0)),
            pl.BlockSpec((1, D), lambda i: (0, 0)),
        ],
        out_specs=[
            pl.BlockSpec((_BLK, D), lambda i: (i, 0)),
            pl.BlockSpec((_BLK, 1), lambda i: (i, 0)),
        ],
        out_shape=[
            jax.ShapeDtypeStruct((N, D), jnp.float32),
            jax.ShapeDtypeStruct((N, 1), jnp.float32),
        ],
    )(nt3, counts[0], counts[1], tab, wt, wb, b.reshape(1, D))


def _dense_layer_body(x_ref, p0_ref, p1_ref, inv_ref, wt_ref, wb_ref, b_ref,
                      o_ref):
    agg = (p0_ref[...] + p1_ref[...]) * inv_ref[...]
    acc = jnp.dot(x_ref[...], wt_ref[...], preferred_element_type=jnp.float32)
    acc += jnp.dot(agg, wb_ref[...], preferred_element_type=jnp.float32)
    o_ref[...] = jnp.maximum(acc + b_ref[...], 0.0)


def _dense_layer(x, partials, inv_deg, W, b):
    wt, wb = W[:D], W[D:]
    return pl.pallas_call(
        _dense_layer_body,
        grid=(N // _BLK,),
        in_specs=[
            pl.BlockSpec((_BLK, D), lambda i: (i, 0)),
            pl.BlockSpec((_BLK, D), lambda i: (i, 0)),
            pl.BlockSpec((_BLK, D), lambda i: (i, 0)),
            pl.BlockSpec((_BLK, 1), lambda i: (i, 0)),
            pl.BlockSpec((D, D), lambda i: (0, 0)),
            pl.BlockSpec((D, D), lambda i: (0, 0)),
            pl.BlockSpec((1, D), lambda i: (0, 0)),
        ],
        out_specs=pl.BlockSpec((_BLK, D), lambda i: (i, 0)),
        out_shape=jax.ShapeDtypeStruct((N, D), jnp.float32),
    )(x, partials[0], partials[1], inv_deg, wt, wb, b.reshape(1, D))


def _heads_body(x_ref, wc1_ref, bc1_ref, wc2_ref, bc2_ref, v_ref,
                emb_ref, best_ref):
    i = pl.program_id(0)
    x = x_ref[...]
    h = jnp.maximum(jnp.dot(x, wc1_ref[...], preferred_element_type=jnp.float32)
                    + bc1_ref[...], 0.0)
    v = jnp.dot(h, wc2_ref[...], preferred_element_type=jnp.float32) + bc2_ref[0, 0]
    v = v[:, 0]
    v_ref[0, 0, :] = v

    blk_arg = jnp.argmax(v)
    blk_max = jnp.max(v)

    @pl.when(i == 0)
    def _():
        best_ref[0] = blk_max - 1.0

    prev = best_ref[0]
    take = blk_max > prev

    @pl.when(take)
    def _():
        best_ref[0] = blk_max
        mask = (lax.broadcasted_iota(jnp.int32, (_BLK, 1), 0) == blk_arg
                ).astype(jnp.float32)
        sel = jnp.sum(x * mask, axis=0, keepdims=True)
        emb_ref[...] = jnp.broadcast_to(sel, (8, D))


def _heads(x, Wc1, bc1, Wc2, bc2):
    values, emb = pl.pallas_call(
        _heads_body,
        grid=(N // _BLK,),
        in_specs=[
            pl.BlockSpec((_BLK, D), lambda i: (i, 0)),
            pl.BlockSpec((D, CRIT_H), lambda i: (0, 0)),
            pl.BlockSpec((1, CRIT_H), lambda i: (0, 0)),
            pl.BlockSpec((CRIT_H, 1), lambda i: (0, 0)),
            pl.BlockSpec((1, 1), lambda i: (0, 0)),
        ],
        out_specs=[
            pl.BlockSpec((1, 1, _BLK), lambda i: (i, 0, 0)),
            pl.BlockSpec((8, D), lambda i: (0, 0)),
        ],
        out_shape=[
            jax.ShapeDtypeStruct((N // _BLK, 1, _BLK), jnp.float32),
            jax.ShapeDtypeStruct((8, D), jnp.float32),
        ],
        scratch_shapes=[pltpu.SMEM((1,), jnp.float32)],
    )(x, Wc1, bc1.reshape(1, CRIT_H), Wc2, bc2.reshape(1, 1))
    return values.reshape(N), emb


def _actor_body(emb_ref, wa1_ref, ba1_ref, wa2_ref, ba2_ref, o_ref):
    h = jnp.maximum(jnp.dot(emb_ref[...], wa1_ref[...],
                            preferred_element_type=jnp.float32) + ba1_ref[...], 0.0)
    o_ref[...] = jnp.dot(h, wa2_ref[...],
                         preferred_element_type=jnp.float32) + ba2_ref[...]


def _actor(emb, Wa1, ba1, Wa2, ba2):
    out = pl.pallas_call(
        _actor_body,
        out_shape=jax.ShapeDtypeStruct((8, ADIM), jnp.float32),
    )(emb, Wa1, ba1.reshape(1, ACT_H), Wa2, ba2.reshape(1, ADIM))
    return out[0]


def kernel(node_types, edge_index, embed_table, W0, b0, W1, b1, W2, b2,
           Wc1, bc1, Wc2, bc2, Wa1, ba1, Wa2, ba2):
    nt = node_types.astype(jnp.int32)
    pad = _EPAD - E
    ei_pad = jnp.concatenate(
        [edge_index.astype(jnp.int32),
         jnp.concatenate([jnp.zeros((1, pad), jnp.int32),
                          jnp.full((1, pad), N, jnp.int32)], axis=0)], axis=1)

    counts = _sc_counts(nt, ei_pad)
    x, inv_deg = _layer0(nt, counts, embed_table, W0, b0)
    for W, b in ((W1, b1), (W2, b2)):
        partials = _sc_agg(x, ei_pad)
        x = _dense_layer(x, partials, inv_deg, W, b)
    values, emb = _heads(x, Wc1, bc1, Wc2, bc2)
    xfer = _actor(emb, Wa1, ba1, Wa2, ba2)
    return jnp.concatenate([values, xfer])


# speedup vs baseline: 1.2285x; 1.2285x over previous
import functools

import jax
import jax.numpy as jnp
from jax import lax
from jax.experimental import pallas as pl
from jax.experimental.pallas import tpu as pltpu
from jax.experimental.pallas import tpu_sc as plsc

N = 10000
E = 320000
D = 128
NUM_GATE = 29
ACT_H = 256
CRIT_H = 128
ADIM = 512

_BLK = 1000

_NC = 2
_NS = 16
_NW = _NC * _NS
_CH = 128
_CPW = 79
_EPAD = _NW * _CH * _CPW
_NROWS = 10240
_RPT = _NROWS // _NS
_NTPAD = 10112


def _worker_id():
    c = lax.axis_index("c")
    s = lax.axis_index("s")
    return c, s, c * _NS + s


def _zero_vmem_2d(ref, nrows, width):
    zeros16 = jnp.zeros((16,), jnp.float32)

    def row(i, _):
        for j in range(width // 16):
            ref[i, pl.ds(j * 16, 16)] = zeros16
        return 0

    lax.fori_loop(0, nrows, row, 0)


def _zero_spmem_slice(zb_ref, sh_ref, base, total):
    off = 0
    while off < total:
        n = min(128, total - off)
        pltpu.sync_copy(zb_ref.at[pl.ds(0, n)], sh_ref.at[pl.ds(base + off, n)])
        off += n


def _zero_spmem_slice_async(zb_ref, sh_ref, base, total, sem):
    cps = []
    off = 0
    while off < total:
        n = min(128, total - off)
        cps.append(pltpu.async_copy(zb_ref.at[pl.ds(0, n)],
                                    sh_ref.at[pl.ds(base + off, n)], sem))
        off += n
    return cps


def _sc_counts_body(nt_hbm, ei_hbm, out_hbm, nt_v, src_v, dst_v, oh_v, idx_v,
                    zb_v, counts_sh, sem):
    c, s, w = _worker_id()
    lane = lax.broadcasted_iota(jnp.int32, (16,), 0)
    ones16 = jnp.full((16,), 1.0, jnp.float32)
    zeros16 = jnp.zeros((16,), jnp.float32)

    pltpu.sync_copy(nt_hbm, nt_v.at[pl.ds(0, N)])
    _zero_vmem_2d(oh_v, _CH, D)
    _zero_vmem_2d(zb_v, 128, D)
    _zero_spmem_slice(zb_v, counts_sh, s * _RPT, _RPT)
    plsc.subcore_barrier()

    def chunk(j, _):
        base = (w * _CPW + j) * _CH
        pltpu.sync_copy(ei_hbm.at[0, pl.ds(base, _CH)], src_v)
        pltpu.sync_copy(ei_hbm.at[1, pl.ds(base, _CH)], dst_v)
        for k in range(_CH // 16):
            s16 = src_v[pl.ds(k * 16, 16)]
            t16 = plsc.load_gather(nt_v, [s16])
            row16 = lane + (k * 16)
            plsc.store_scatter(oh_v, [row16, t16], ones16)
            idx_v[pl.ds(k * 16, 16)] = t16
        pltpu.sync_copy(oh_v, counts_sh.at[dst_v], add=True)
        for k in range(_CH // 16):
            t16 = idx_v[pl.ds(k * 16, 16)]
            row16 = lane + (k * 16)
            plsc.store_scatter(oh_v, [row16, t16], zeros16)
        return 0

    lax.fori_loop(0, _CPW, chunk, 0)
    plsc.subcore_barrier()
    row0 = s * _RPT
    pltpu.sync_copy(counts_sh.at[pl.ds(row0, _RPT)],
                    out_hbm.at[c, pl.ds(row0, _RPT)])


def _sc_counts(node_types, ei_pad):
    mesh = plsc.VectorSubcoreMesh(core_axis_name="c", subcore_axis_name="s")
    f = pl.kernel(
        _sc_counts_body,
        out_type=jax.ShapeDtypeStruct((_NC, _NROWS, D), jnp.float32),
        mesh=mesh,
        compiler_params=pltpu.CompilerParams(needs_layout_passes=False),
        scratch_types=[
            pltpu.VMEM((_NTPAD,), jnp.int32),
            pltpu.VMEM((_CH,), jnp.int32),
            pltpu.VMEM((_CH,), jnp.int32),
            pltpu.VMEM((_CH, D), jnp.float32),
            pltpu.VMEM((_CH,), jnp.int32),
            pltpu.VMEM((128, D), jnp.float32),
            pltpu.VMEM_SHARED((_NROWS, D), jnp.float32),
            pltpu.SemaphoreType.DMA,
        ],
    )
    return f(node_types, ei_pad)


def _sc_agg_body(x_hbm, ei_hbm, out_hbm, src_v, dst_v, rows_v, zb_v,
                 agg_sh, sem):
    c, s, w = _worker_id()
    _zero_vmem_2d(zb_v, 128, D)
    _zero_spmem_slice(zb_v, agg_sh, s * _RPT, _RPT)
    plsc.subcore_barrier()

    def chunk(j, _):
        base = (w * _CPW + j) * _CH
        pltpu.sync_copy(ei_hbm.at[0, pl.ds(base, _CH)], src_v)
        pltpu.sync_copy(ei_hbm.at[1, pl.ds(base, _CH)], dst_v)
        pltpu.async_copy(x_hbm.at[src_v], rows_v, sem).wait()
        pltpu.sync_copy(rows_v, agg_sh.at[dst_v], add=True)
        return 0

    lax.fori_loop(0, _CPW, chunk, 0)
    plsc.subcore_barrier()
    row0 = s * _RPT
    pltpu.sync_copy(agg_sh.at[pl.ds(row0, _RPT)],
                    out_hbm.at[c, pl.ds(row0, _RPT)])


def _sc_agg(x, ei_pad):
    mesh = plsc.VectorSubcoreMesh(core_axis_name="c", subcore_axis_name="s")
    f = pl.kernel(
        _sc_agg_body,
        out_type=jax.ShapeDtypeStruct((_NC, _NROWS, D), jnp.float32),
        mesh=mesh,
        compiler_params=pltpu.CompilerParams(needs_layout_passes=False),
        scratch_types=[
            pltpu.VMEM((_CH,), jnp.int32),
            pltpu.VMEM((_CH,), jnp.int32),
            pltpu.VMEM((_CH, D), jnp.float32),
            pltpu.VMEM((128, D), jnp.float32),
            pltpu.VMEM_SHARED((_NROWS, D), jnp.float32),
            pltpu.SemaphoreType.DMA,
        ],
    )
    return f(x, ei_pad)


def _layer0_body(nt_ref, c0_ref, c1_ref, tab_ref, wt_ref, wb_ref, b_ref,
                 o_ref, inv_ref):
    counts = c0_ref[...] + c1_ref[...]
    deg = jnp.maximum(jnp.sum(counts, axis=1), 1.0)
    inv = 1.0 / deg
    tab = tab_ref[...]
    types_row = nt_ref[0, ...]
    gates = lax.broadcasted_iota(jnp.int32, (D, _BLK), 0)
    onehot_t = (gates == types_row).astype(jnp.float32)
    x0 = lax.dot_general(onehot_t, tab, (((0,), (0,)), ((), ())),
                         preferred_element_type=jnp.float32)
    agg0 = jnp.dot(counts, tab, preferred_element_type=jnp.float32)
    agg0 = agg0 * inv[:, None]
    acc = jnp.dot(x0, wt_ref[...], preferred_element_type=jnp.float32)
    acc += jnp.dot(agg0, wb_ref[...], preferred_element_type=jnp.float32)
    o_ref[...] = jnp.maximum(acc + b_ref[...], 0.0)
    inv_ref[...] = inv[:, None]


def _layer0(node_types, counts, embed_table, W, b):
    wt, wb = W[:D], W[D:]
    tab = jnp.zeros((D, D), jnp.float32).at[:NUM_GATE].set(embed_table)
    nt3 = node_types.astype(jnp.int32).reshape(N // _BLK, 1, _BLK)
    return pl.pallas_call(
        _layer0_body,
        grid=(N // _BLK,),
        in_specs=[
            pl.BlockSpec((1, 1, _BLK), lambda i: (i, 0, 0)),
            pl.BlockSpec((_BLK, D), lambda i: (i, 0)),
            pl.BlockSpec((_BLK, D), lambda i: (i, 0)),
            pl.BlockSpec((D, D), lambda i: (0, 0)),
            pl.BlockSpec((D, D), lambda i: (0, 0)),
            pl.BlockSpec((D, D), lambda i: (0, 0)),
            pl.BlockSpec((1, D), lambda i: (0, 0)),
        ],
        out_specs=[
            pl.BlockSpec((_BLK, D), lambda i: (i, 0)),
            pl.BlockSpec((_BLK, 1), lambda i: (i, 0)),
        ],
        out_shape=[
            jax.ShapeDtypeStruct((N, D), jnp.float32),
            jax.ShapeDtypeStruct((N, 1), jnp.float32),
        ],
    )(nt3, counts[0], counts[1], tab, wt, wb, b.reshape(1, D))


def _dense_layer_body(x_ref, p0_ref, p1_ref, inv_ref, wt_ref, wb_ref, b_ref,
                      o_ref):
    agg = (p0_ref[...] + p1_ref[...]) * inv_ref[...]
    acc = jnp.dot(x_ref[...], wt_ref[...], preferred_element_type=jnp.float32)
    acc += jnp.dot(agg, wb_ref[...], preferred_element_type=jnp.float32)
    o_ref[...] = jnp.maximum(acc + b_ref[...], 0.0)


def _dense_layer(x, partials, inv_deg, W, b):
    wt, wb = W[:D], W[D:]
    return pl.pallas_call(
        _dense_layer_body,
        grid=(N // _BLK,),
        in_specs=[
            pl.BlockSpec((_BLK, D), lambda i: (i, 0)),
            pl.BlockSpec((_BLK, D), lambda i: (i, 0)),
            pl.BlockSpec((_BLK, D), lambda i: (i, 0)),
            pl.BlockSpec((_BLK, 1), lambda i: (i, 0)),
            pl.BlockSpec((D, D), lambda i: (0, 0)),
            pl.BlockSpec((D, D), lambda i: (0, 0)),
            pl.BlockSpec((1, D), lambda i: (0, 0)),
        ],
        out_specs=pl.BlockSpec((_BLK, D), lambda i: (i, 0)),
        out_shape=jax.ShapeDtypeStruct((N, D), jnp.float32),
    )(x, partials[0], partials[1], inv_deg, wt, wb, b.reshape(1, D))


def _heads_body(x_ref, wc1_ref, bc1_ref, wc2_ref, bc2_ref, v_ref,
                emb_ref, best_ref):
    i = pl.program_id(0)
    x = x_ref[...]
    h = jnp.maximum(jnp.dot(x, wc1_ref[...], preferred_element_type=jnp.float32)
                    + bc1_ref[...], 0.0)
    v = jnp.dot(h, wc2_ref[...], preferred_element_type=jnp.float32) + bc2_ref[0, 0]
    v = v[:, 0]
    v_ref[0, 0, :] = v

    blk_arg = jnp.argmax(v)
    blk_max = jnp.max(v)

    @pl.when(i == 0)
    def _():
        best_ref[0] = blk_max - 1.0

    prev = best_ref[0]
    take = blk_max > prev

    @pl.when(take)
    def _():
        best_ref[0] = blk_max
        mask = (lax.broadcasted_iota(jnp.int32, (_BLK, 1), 0) == blk_arg
                ).astype(jnp.float32)
        sel = jnp.sum(x * mask, axis=0, keepdims=True)
        emb_ref[...] = jnp.broadcast_to(sel, (8, D))


def _heads(x, Wc1, bc1, Wc2, bc2):
    values, emb = pl.pallas_call(
        _heads_body,
        grid=(N // _BLK,),
        in_specs=[
            pl.BlockSpec((_BLK, D), lambda i: (i, 0)),
            pl.BlockSpec((D, CRIT_H), lambda i: (0, 0)),
            pl.BlockSpec((1, CRIT_H), lambda i: (0, 0)),
            pl.BlockSpec((CRIT_H, 1), lambda i: (0, 0)),
            pl.BlockSpec((1, 1), lambda i: (0, 0)),
        ],
        out_specs=[
            pl.BlockSpec((1, 1, _BLK), lambda i: (i, 0, 0)),
            pl.BlockSpec((8, D), lambda i: (0, 0)),
        ],
        out_shape=[
            jax.ShapeDtypeStruct((N // _BLK, 1, _BLK), jnp.float32),
            jax.ShapeDtypeStruct((8, D), jnp.float32),
        ],
        scratch_shapes=[pltpu.SMEM((1,), jnp.float32)],
    )(x, Wc1, bc1.reshape(1, CRIT_H), Wc2, bc2.reshape(1, 1))
    return values.reshape(N), emb


def _actor_body(emb_ref, wa1_ref, ba1_ref, wa2_ref, ba2_ref, o_ref):
    h = jnp.maximum(jnp.dot(emb_ref[...], wa1_ref[...],
                            preferred_element_type=jnp.float32) + ba1_ref[...], 0.0)
    o_ref[...] = jnp.dot(h, wa2_ref[...],
                         preferred_element_type=jnp.float32) + ba2_ref[...]


def _actor(emb, Wa1, ba1, Wa2, ba2):
    out = pl.pallas_call(
        _actor_body,
        out_shape=jax.ShapeDtypeStruct((8, ADIM), jnp.float32),
    )(emb, Wa1, ba1.reshape(1, ACT_H), Wa2, ba2.reshape(1, ADIM))
    return out[0]


def kernel(node_types, edge_index, embed_table, W0, b0, W1, b1, W2, b2,
           Wc1, bc1, Wc2, bc2, Wa1, ba1, Wa2, ba2):
    nt = node_types.astype(jnp.int32)
    pad = _EPAD - E
    ei_pad = jnp.concatenate(
        [edge_index.astype(jnp.int32),
         jnp.concatenate([jnp.zeros((1, pad), jnp.int32),
                          jnp.full((1, pad), N, jnp.int32)], axis=0)], axis=1)

    counts = _sc_counts(nt, ei_pad)
    x, inv_deg = _layer0(nt, counts, embed_table, W0, b0)
    for W, b in ((W1, b1), (W2, b2)):
        partials = _sc_agg(x, ei_pad)
        x = _dense_layer(x, partials, inv_deg, W, b)
    values, emb = _heads(x, Wc1, bc1, Wc2, bc2)
    xfer = _actor(emb, Wa1, ba1, Wa2, ba2)
    return jnp.concatenate([values, xfer])


# speedup vs baseline: 1.2289x; 1.0003x over previous
import jax
import jax.numpy as jnp
from jax import lax
from jax.experimental import pallas as pl
from jax.experimental.pallas import tpu as pltpu
from jax.experimental.pallas import tpu_sc as plsc

N = 10000
E = 320000
D = 128
NUM_GATE = 29
ACT_H = 256
CRIT_H = 128
ADIM = 512

_BLK = 1000

_NC = 2
_NS = 16
_NW = _NC * _NS
_CH = 128
_CPW = 79
_EPAD = _NW * _CH * _CPW
_NROWS = 10240
_RPT = _NROWS // _NS
_NTPAD = 10112


def _worker_id():
    c = lax.axis_index("c")
    s = lax.axis_index("s")
    return c, s, c * _NS + s


def _zero_vmem_2d(ref, nrows, width):
    zeros16 = jnp.zeros((16,), jnp.float32)

    def row(i, _):
        for j in range(width // 16):
            ref[i, pl.ds(j * 16, 16)] = zeros16
        return 0

    lax.fori_loop(0, nrows, row, 0)


def _zero_spmem_slice(zb_ref, sh_ref, base, total):
    off = 0
    while off < total:
        n = min(128, total - off)
        pltpu.sync_copy(zb_ref.at[pl.ds(0, n)], sh_ref.at[pl.ds(base + off, n)])
        off += n


def _sc_counts_body(nt_hbm, ei_hbm, out_hbm, nt_v, src_v, dst_v, oh_v, idx_v,
                    zb_v, counts_sh, sem):
    c, s, w = _worker_id()
    lane = lax.broadcasted_iota(jnp.int32, (16,), 0)
    ones16 = jnp.full((16,), 1.0, jnp.float32)
    zeros16 = jnp.zeros((16,), jnp.float32)

    pltpu.sync_copy(nt_hbm, nt_v.at[pl.ds(0, N)])
    _zero_vmem_2d(oh_v, _CH, D)
    _zero_vmem_2d(zb_v, 128, D)
    _zero_spmem_slice(zb_v, counts_sh, s * _RPT, _RPT)
    plsc.subcore_barrier()

    def chunk(j, _):
        base = (w * _CPW + j) * _CH
        pltpu.sync_copy(ei_hbm.at[0, pl.ds(base, _CH)], src_v)
        pltpu.sync_copy(ei_hbm.at[1, pl.ds(base, _CH)], dst_v)
        for k in range(_CH // 16):
            s16 = src_v[pl.ds(k * 16, 16)]
            t16 = plsc.load_gather(nt_v, [s16])
            row16 = lane + (k * 16)
            plsc.store_scatter(oh_v, [row16, t16], ones16)
            idx_v[pl.ds(k * 16, 16)] = t16
        pltpu.sync_copy(oh_v, counts_sh.at[dst_v], add=True)
        for k in range(_CH // 16):
            t16 = idx_v[pl.ds(k * 16, 16)]
            row16 = lane + (k * 16)
            plsc.store_scatter(oh_v, [row16, t16], zeros16)
        return 0

    lax.fori_loop(0, _CPW, chunk, 0)
    plsc.subcore_barrier()
    row0 = s * _RPT
    pltpu.sync_copy(counts_sh.at[pl.ds(row0, _RPT)],
                    out_hbm.at[c, pl.ds(row0, _RPT)])


def _sc_counts(node_types, ei_pad):
    mesh = plsc.VectorSubcoreMesh(core_axis_name="c", subcore_axis_name="s")
    f = pl.kernel(
        _sc_counts_body,
        out_type=jax.ShapeDtypeStruct((_NC, _NROWS, D), jnp.float32),
        mesh=mesh,
        compiler_params=pltpu.CompilerParams(needs_layout_passes=False),
        scratch_types=[
            pltpu.VMEM((_NTPAD,), jnp.int32),
            pltpu.VMEM((_CH,), jnp.int32),
            pltpu.VMEM((_CH,), jnp.int32),
            pltpu.VMEM((_CH, D), jnp.float32),
            pltpu.VMEM((_CH,), jnp.int32),
            pltpu.VMEM((128, D), jnp.float32),
            pltpu.VMEM_SHARED((_NROWS, D), jnp.float32),
            pltpu.SemaphoreType.DMA,
        ],
    )
    return f(node_types, ei_pad)


def _sc_agg_body(x_hbm, ei_hbm, out_hbm, src_v, dst_v, rows_v, zb_v,
                 agg_sh, sem):
    c, s, w = _worker_id()
    _zero_vmem_2d(zb_v, 128, D)
    _zero_spmem_slice(zb_v, agg_sh, s * _RPT, _RPT)
    plsc.subcore_barrier()

    def chunk(j, _):
        base = (w * _CPW + j) * _CH
        pltpu.sync_copy(ei_hbm.at[0, pl.ds(base, _CH)], src_v)
        pltpu.sync_copy(ei_hbm.at[1, pl.ds(base, _CH)], dst_v)
        pltpu.async_copy(x_hbm.at[src_v], rows_v, sem).wait()
        pltpu.sync_copy(rows_v, agg_sh.at[dst_v], add=True)
        return 0

    lax.fori_loop(0, _CPW, chunk, 0)
    plsc.subcore_barrier()
    row0 = s * _RPT
    pltpu.sync_copy(agg_sh.at[pl.ds(row0, _RPT)],
                    out_hbm.at[c, pl.ds(row0, _RPT)])


def _sc_agg(x, ei_pad):
    mesh = plsc.VectorSubcoreMesh(core_axis_name="c", subcore_axis_name="s")
    f = pl.kernel(
        _sc_agg_body,
        out_type=jax.ShapeDtypeStruct((_NC, _NROWS, D), jnp.float32),
        mesh=mesh,
        compiler_params=pltpu.CompilerParams(needs_layout_passes=False),
        scratch_types=[
            pltpu.VMEM((_CH,), jnp.int32),
            pltpu.VMEM((_CH,), jnp.int32),
            pltpu.VMEM((_CH, D), jnp.float32),
            pltpu.VMEM((128, D), jnp.float32),
            pltpu.VMEM_SHARED((_NROWS, D), jnp.float32),
            pltpu.SemaphoreType.DMA,
        ],
    )
    return f(x, ei_pad)


def _layer0_body(nt_ref, c0_ref, c1_ref, tab_ref, wt_ref, wb_ref, b_ref,
                 o_ref, inv_ref):
    counts = c0_ref[...] + c1_ref[...]
    deg = jnp.maximum(jnp.sum(counts, axis=1), 1.0)
    inv = 1.0 / deg
    tab = tab_ref[...]
    types_row = nt_ref[0, ...]
    gates = lax.broadcasted_iota(jnp.int32, (D, _BLK), 0)
    onehot_t = (gates == types_row).astype(jnp.float32)
    x0 = lax.dot_general(onehot_t, tab, (((0,), (0,)), ((), ())),
                         preferred_element_type=jnp.float32)
    agg0 = jnp.dot(counts, tab, preferred_element_type=jnp.float32)
    agg0 = agg0 * inv[:, None]
    acc = jnp.dot(x0, wt_ref[...], preferred_element_type=jnp.float32)
    acc += jnp.dot(agg0, wb_ref[...], preferred_element_type=jnp.float32)
    o_ref[...] = jnp.maximum(acc + b_ref[...], 0.0)
    inv_ref[...] = inv[:, None]


def _layer0(node_types, counts, embed_table, W, b):
    wt, wb = W[:D], W[D:]
    tab = jnp.zeros((D, D), jnp.float32).at[:NUM_GATE].set(embed_table)
    nt3 = node_types.astype(jnp.int32).reshape(N // _BLK, 1, _BLK)
    return pl.pallas_call(
        _layer0_body,
        grid=(N // _BLK,),
        in_specs=[
            pl.BlockSpec((1, 1, _BLK), lambda i: (i, 0, 0)),
            pl.BlockSpec((_BLK, D), lambda i: (i, 0)),
            pl.BlockSpec((_BLK, D), lambda i: (i, 0)),
            pl.BlockSpec((D, D), lambda i: (0, 0)),
            pl.BlockSpec((D, D), lambda i: (0, 0)),
            pl.BlockSpec((D, D), lambda i: (0, 0)),
            pl.BlockSpec((1, D), lambda i: (0, 0)),
        ],
        out_specs=[
            pl.BlockSpec((_BLK, D), lambda i: (i, 0)),
            pl.BlockSpec((_BLK, 1), lambda i: (i, 0)),
        ],
        out_shape=[
            jax.ShapeDtypeStruct((N, D), jnp.float32),
            jax.ShapeDtypeStruct((N, 1), jnp.float32),
        ],
    )(nt3, counts[0], counts[1], tab, wt, wb, b.reshape(1, D))


def _dense_layer_body(x_ref, p0_ref, p1_ref, inv_ref, wt_ref, wb_ref, b_ref,
                      o_ref):
    agg = (p0_ref[...] + p1_ref[...]) * inv_ref[...]
    acc = jnp.dot(x_ref[...], wt_ref[...], preferred_element_type=jnp.float32)
    acc += jnp.dot(agg, wb_ref[...], preferred_element_type=jnp.float32)
    o_ref[...] = jnp.maximum(acc + b_ref[...], 0.0)


def _dense_layer(x, partials, inv_deg, W, b):
    wt, wb = W[:D], W[D:]
    return pl.pallas_call(
        _dense_layer_body,
        grid=(N // _BLK,),
        in_specs=[
            pl.BlockSpec((_BLK, D), lambda i: (i, 0)),
            pl.BlockSpec((_BLK, D), lambda i: (i, 0)),
            pl.BlockSpec((_BLK, D), lambda i: (i, 0)),
            pl.BlockSpec((_BLK, 1), lambda i: (i, 0)),
            pl.BlockSpec((D, D), lambda i: (0, 0)),
            pl.BlockSpec((D, D), lambda i: (0, 0)),
            pl.BlockSpec((1, D), lambda i: (0, 0)),
        ],
        out_specs=pl.BlockSpec((_BLK, D), lambda i: (i, 0)),
        out_shape=jax.ShapeDtypeStruct((N, D), jnp.float32),
    )(x, partials[0], partials[1], inv_deg, wt, wb, b.reshape(1, D))


def _heads_body(x_ref, wc1_ref, bc1_ref, wc2_ref, bc2_ref, v_ref,
                emb_ref, best_ref):
    i = pl.program_id(0)
    x = x_ref[...]
    h = jnp.maximum(jnp.dot(x, wc1_ref[...], preferred_element_type=jnp.float32)
                    + bc1_ref[...], 0.0)
    v = jnp.dot(h, wc2_ref[...], preferred_element_type=jnp.float32) + bc2_ref[0, 0]
    v = v[:, 0]
    v_ref[0, 0, :] = v

    blk_arg = jnp.argmax(v)
    blk_max = jnp.max(v)

    @pl.when(i == 0)
    def _():
        best_ref[0] = blk_max - 1.0

    prev = best_ref[0]
    take = blk_max > prev

    @pl.when(take)
    def _():
        best_ref[0] = blk_max
        mask = (lax.broadcasted_iota(jnp.int32, (_BLK, 1), 0) == blk_arg
                ).astype(jnp.float32)
        sel = jnp.sum(x * mask, axis=0, keepdims=True)
        emb_ref[...] = jnp.broadcast_to(sel, (8, D))


def _heads(x, Wc1, bc1, Wc2, bc2):
    values, emb = pl.pallas_call(
        _heads_body,
        grid=(N // _BLK,),
        in_specs=[
            pl.BlockSpec((_BLK, D), lambda i: (i, 0)),
            pl.BlockSpec((D, CRIT_H), lambda i: (0, 0)),
            pl.BlockSpec((1, CRIT_H), lambda i: (0, 0)),
            pl.BlockSpec((CRIT_H, 1), lambda i: (0, 0)),
            pl.BlockSpec((1, 1), lambda i: (0, 0)),
        ],
        out_specs=[
            pl.BlockSpec((1, 1, _BLK), lambda i: (i, 0, 0)),
            pl.BlockSpec((8, D), lambda i: (0, 0)),
        ],
        out_shape=[
            jax.ShapeDtypeStruct((N // _BLK, 1, _BLK), jnp.float32),
            jax.ShapeDtypeStruct((8, D), jnp.float32),
        ],
        scratch_shapes=[pltpu.SMEM((1,), jnp.float32)],
    )(x, Wc1, bc1.reshape(1, CRIT_H), Wc2, bc2.reshape(1, 1))
    return values.reshape(N), emb


def _actor_body(emb_ref, wa1_ref, ba1_ref, wa2_ref, ba2_ref, o_ref):
    h = jnp.maximum(jnp.dot(emb_ref[...], wa1_ref[...],
                            preferred_element_type=jnp.float32) + ba1_ref[...], 0.0)
    o_ref[...] = jnp.dot(h, wa2_ref[...],
                         preferred_element_type=jnp.float32) + ba2_ref[...]


def _actor(emb, Wa1, ba1, Wa2, ba2):
    out = pl.pallas_call(
        _actor_body,
        out_shape=jax.ShapeDtypeStruct((8, ADIM), jnp.float32),
    )(emb, Wa1, ba1.reshape(1, ACT_H), Wa2, ba2.reshape(1, ADIM))
    return out[0]


def kernel(node_types, edge_index, embed_table, W0, b0, W1, b1, W2, b2,
           Wc1, bc1, Wc2, bc2, Wa1, ba1, Wa2, ba2):
    nt = node_types.astype(jnp.int32)
    pad = _EPAD - E
    ei_pad = jnp.concatenate(
        [edge_index.astype(jnp.int32),
         jnp.concatenate([jnp.zeros((1, pad), jnp.int32),
                          jnp.full((1, pad), N, jnp.int32)], axis=0)], axis=1)

    counts = _sc_counts(nt, ei_pad)
    x, inv_deg = _layer0(nt, counts, embed_table, W0, b0)
    for W, b in ((W1, b1), (W2, b2)):
        partials = _sc_agg(x, ei_pad)
        x = _dense_layer(x, partials, inv_deg, W, b)
    values, emb = _heads(x, Wc1, bc1, Wc2, bc2)
    xfer = _actor(emb, Wa1, ba1, Wa2, ba2)
    return jnp.concatenate([values, xfer])
